# Initial kernel scaffold; baseline (speedup 1.0000x reference)
import jax
import jax.numpy as jnp
from jax.experimental import pallas as pl


def kernel(x, centroids):
    raise NotImplementedError("write your pallas kernel here")


# speedup vs baseline: 1.2809x; 1.2809x over previous
import functools

import jax
import jax.numpy as jnp
from jax import lax
from jax.experimental import pallas as pl
from jax.experimental.pallas import tpu as pltpu
from jax.experimental.pallas import tpu_sc as plsc

_N = 131072
_F = 32
_K = 512
_ITER = 10
_B = 2048
_NBLK = _N // _B

_R = 48
_NW = 32
_HALF = _N // 2
_WIN = 960
_PAD = 1024


def _chunk_lo(j):
    return jnp.where(j <= 11, 4224 * j, 46464 + 3840 * (j - 11))


def _assign_body(xb_ref, cb_ref, b2_ref, out_ref):
    ab = jax.lax.dot_general(xb_ref[:, :], cb_ref[:, :],
                             (((1,), (1,)), ((), ())),
                             preferred_element_type=jnp.float32)
    d2 = b2_ref[:, :] - 2.0 * ab
    m = jnp.min(d2, axis=1, keepdims=True)
    lane = jax.lax.broadcasted_iota(jnp.int32, (_B, _K), 1)
    idx = jnp.min(jnp.where(d2 <= m, lane, _K), axis=1)
    out_ref[0, 0, :] = idx


_assign = pl.pallas_call(
    _assign_body,
    grid=(_NBLK,),
    in_specs=[
        pl.BlockSpec((_B, _F), lambda b: (b, 0)),
        pl.BlockSpec((_K, _F), lambda b: (0, 0)),
        pl.BlockSpec((1, _K), lambda b: (0, 0)),
    ],
    out_specs=pl.BlockSpec((1, 1, _B), lambda b: (b, 0, 0)),
    out_shape=jax.ShapeDtypeStruct((_NBLK, 1, _B), jnp.int32),
)


def _reduce_body(xs_ref, ks_ref, out_ref, xwin, kwin, acc):
    info = plsc.get_sparse_core_info()
    nc = info.num_cores
    wid = lax.axis_index("s") * nc + lax.axis_index("c")
    half = wid // 16
    j = wid % 16
    lo = half * _HALF + _chunk_lo(j)
    length = jnp.where(j < 15, _chunk_lo(j + 1), _HALF) - _chunk_lo(j)

    zeros16 = jnp.zeros((16,), jnp.float32)

    def zero_body(i, _):
        acc[pl.ds(i * 16, 16)] = zeros16
        return 0

    lax.fori_loop(0, (_K * _R) // 16, zero_body, 0)

    nwin = (length + _WIN - 1) // _WIN

    def win_body(w, _):
        start = lo + w * _WIN
        pltpu.sync_copy(xs_ref.at[pl.ds(start * _R, _WIN * _R)], xwin)
        pltpu.sync_copy(ks_ref.at[pl.ds(start, _WIN)], kwin)
        rows = jnp.minimum(_WIN, length - w * _WIN)

        def grp_body(g, _):
            kvec = kwin[pl.ds(g * 16, 16)]
            for t in range(16):
                k = kvec[t]
                base = k * _R
                roff = (g * 16 + t) * _R
                for h in range(_R // 16):
                    av = acc[pl.ds(base + h * 16, 16)]
                    xv = xwin[pl.ds(roff + h * 16, 16)]
                    acc[pl.ds(base + h * 16, 16)] = av + xv
            return 0

        lax.fori_loop(0, rows // 16, grp_body, 0)
        return 0

    lax.fori_loop(0, nwin, win_body, 0)
    pltpu.sync_copy(acc, out_ref.at[wid])


_reduce = functools.partial(
    pl.kernel,
    mesh=plsc.VectorSubcoreMesh(core_axis_name="c", subcore_axis_name="s"),
    out_type=jax.ShapeDtypeStruct((_NW, _K * _R), jnp.float32),
    scratch_types=[
        pltpu.VMEM((_WIN * _R,), jnp.float32),
        pltpu.VMEM((_WIN,), jnp.int32),
        pltpu.VMEM((_K * _R,), jnp.float32),
    ],
)(_reduce_body)


def _fold_body(planes_ref, c_ref, cnew_ref):
    s = planes_ref[0, :, :]
    for c in range(1, _NW):
        s = s + planes_ref[c, :, :]
    counts = s[:, _F:_F + 1]
    sums = s[:, 0:_F]
    means = sums / jnp.maximum(counts, 1.0)
    cnew_ref[:, :] = jnp.where(counts > 0.0, means, c_ref[:, :])


_fold = pl.pallas_call(
    _fold_body,
    in_specs=[
        pl.BlockSpec((_NW, _K, _R), lambda: (0, 0, 0)),
        pl.BlockSpec((_K, _F), lambda: (0, 0)),
    ],
    out_specs=pl.BlockSpec((_K, _F), lambda: (0, 0)),
    out_shape=jax.ShapeDtypeStruct((_K, _F), jnp.float32),
)


@jax.jit
def kernel(x, centroids):
    xb = x.astype(jnp.bfloat16)
    xpad = jnp.concatenate(
        [x, jnp.ones((_N, 1), jnp.float32),
         jnp.zeros((_N, _R - _F - 1), jnp.float32)], axis=1)
    c = centroids
    asg = None
    for it in range(_ITER):
        cb = c.astype(jnp.bfloat16)
        b2 = jnp.sum(c * c, axis=1)[None, :]
        asg = _assign(xb, cb, b2).reshape(_N)
        if it < _ITER - 1:
            perm = jnp.argsort(asg)
            xs = jnp.concatenate(
                [xpad[perm], jnp.zeros((_PAD, _R), jnp.float32)], axis=0)
            ks = jnp.concatenate(
                [asg[perm], jnp.zeros((_PAD,), jnp.int32)], axis=0)
            planes = _reduce(xs.reshape(-1), ks)
            c = _fold(planes.reshape(_NW, _K, _R), c)
    return asg


# speedup vs baseline: 1.4084x; 1.0995x over previous
import functools

import jax
import jax.numpy as jnp
from jax import lax
from jax.experimental import pallas as pl
from jax.experimental.pallas import tpu as pltpu
from jax.experimental.pallas import tpu_sc as plsc

_N = 131072
_F = 32
_K = 512
_ITER = 10
_B = 2048
_NBLK = _N // _B

_R = 48
_NW = 32
_HALF = _N // 2
_WIN = 960
_PAD = 1024


def _chunk_lo(j):
    return jnp.where(j <= 11, 4224 * j, 46464 + 3840 * (j - 11))


def _assign_body(xb_ref, cb_ref, b2_ref, out_ref):
    ab = jax.lax.dot_general(xb_ref[:, :], cb_ref[:, :],
                             (((1,), (1,)), ((), ())),
                             preferred_element_type=jnp.float32)
    d2 = b2_ref[:, :] - 2.0 * ab
    m = jnp.min(d2, axis=1, keepdims=True)
    lane = jax.lax.broadcasted_iota(jnp.int32, (_B, _K), 1)
    idx = jnp.min(jnp.where(d2 <= m, lane, _K), axis=1)
    out_ref[0, 0, :] = idx


_assign = pl.pallas_call(
    _assign_body,
    grid=(_NBLK,),
    in_specs=[
        pl.BlockSpec((_B, _F), lambda b: (b, 0)),
        pl.BlockSpec((_K, _F), lambda b: (0, 0)),
        pl.BlockSpec((1, _K), lambda b: (0, 0)),
    ],
    out_specs=pl.BlockSpec((1, 1, _B), lambda b: (b, 0, 0)),
    out_shape=jax.ShapeDtypeStruct((_NBLK, 1, _B), jnp.int32),
)


def _reduce_body(xs_ref, ks_ref, out_ref, xwin, kwin, acc):
    info = plsc.get_sparse_core_info()
    nc = info.num_cores
    wid = lax.axis_index("s") * nc + lax.axis_index("c")
    half = wid // 16
    j = wid % 16
    lo = half * _HALF + _chunk_lo(j)
    length = jnp.where(j < 15, _chunk_lo(j + 1), _HALF) - _chunk_lo(j)

    zeros16 = jnp.zeros((16,), jnp.float32)
    one16 = jnp.where(lax.iota(jnp.int32, 16) == 0, 1.0, 0.0)

    def zero_body(i, _):
        acc[pl.ds(i * 16, 16)] = zeros16
        return 0

    lax.fori_loop(0, (_K * _R) // 16, zero_body, 0)

    nwin = (length + _WIN - 1) // _WIN

    def win_body(w, _):
        start = lo + w * _WIN
        pltpu.sync_copy(xs_ref.at[pl.ds(start * _F, _WIN * _F)], xwin)
        pltpu.sync_copy(ks_ref.at[pl.ds(start, _WIN)], kwin)
        rows = jnp.minimum(_WIN, length - w * _WIN)

        def grp_body(g, _):
            kvec = kwin[pl.ds(g * 16, 16)]
            for t in range(16):
                k = kvec[t]
                base = k * _R
                roff = (g * 16 + t) * _F
                for h in range(_F // 16):
                    av = acc[pl.ds(base + h * 16, 16)]
                    xv = xwin[pl.ds(roff + h * 16, 16)]
                    acc[pl.ds(base + h * 16, 16)] = av + xv
                cv = acc[pl.ds(base + _F, 16)]
                acc[pl.ds(base + _F, 16)] = cv + one16
            return 0

        lax.fori_loop(0, rows // 16, grp_body, 0)
        return 0

    lax.fori_loop(0, nwin, win_body, 0)
    pltpu.sync_copy(acc, out_ref.at[wid])


_reduce = functools.partial(
    pl.kernel,
    mesh=plsc.VectorSubcoreMesh(core_axis_name="c", subcore_axis_name="s"),
    out_type=jax.ShapeDtypeStruct((_NW, _K * _R), jnp.float32),
    scratch_types=[
        pltpu.VMEM((_WIN * _F,), jnp.float32),
        pltpu.VMEM((_WIN,), jnp.int32),
        pltpu.VMEM((_K * _R,), jnp.float32),
    ],
)(_reduce_body)


def _fold_body(planes_ref, c_ref, cnew_ref):
    s = planes_ref[0, :, :]
    for c in range(1, _NW):
        s = s + planes_ref[c, :, :]
    counts = s[:, _F:_F + 1]
    sums = s[:, 0:_F]
    means = sums / jnp.maximum(counts, 1.0)
    cnew_ref[:, :] = jnp.where(counts > 0.0, means, c_ref[:, :])


_fold = pl.pallas_call(
    _fold_body,
    in_specs=[
        pl.BlockSpec((_NW, _K, _R), lambda: (0, 0, 0)),
        pl.BlockSpec((_K, _F), lambda: (0, 0)),
    ],
    out_specs=pl.BlockSpec((_K, _F), lambda: (0, 0)),
    out_shape=jax.ShapeDtypeStruct((_K, _F), jnp.float32),
)


@jax.jit
def kernel(x, centroids):
    xb = x.astype(jnp.bfloat16)
    iota = jnp.arange(_N, dtype=jnp.int32)
    c = centroids
    asg = None
    for it in range(_ITER):
        cb = c.astype(jnp.bfloat16)
        b2 = jnp.sum(c * c, axis=1)[None, :]
        asg = _assign(xb, cb, b2).reshape(_N)
        if it < _ITER - 1:
            ks_sorted, perm = jax.lax.sort((asg, iota), num_keys=1)
            xs = jnp.concatenate(
                [x[perm], jnp.zeros((_PAD, _F), jnp.float32)], axis=0)
            ksp = jnp.concatenate(
                [ks_sorted, jnp.zeros((_PAD,), jnp.int32)], axis=0)
            planes = _reduce(xs.reshape(-1), ksp)
            c = _fold(planes.reshape(_NW, _K, _R), c)
    return asg


# speedup vs baseline: 1.8484x; 1.3125x over previous
import functools

import jax
import jax.numpy as jnp
from jax import lax
from jax.experimental import pallas as pl
from jax.experimental.pallas import tpu as pltpu
from jax.experimental.pallas import tpu_sc as plsc

_N = 131072
_F = 32
_K = 512
_ITER = 10
_B = 2048
_NBLK = _N // _B

_R = 48
_NW = 32
_HALF = _N // 2
_WIN = 960
_PAD = 1024


def _chunk_lo(j):
    return jnp.where(j <= 11, 4224 * j, 46464 + 3840 * (j - 11))


def _assign_body(xb_ref, cb_ref, b2_ref, out_ref):
    ab = jax.lax.dot_general(xb_ref[:, :], cb_ref[:, :],
                             (((1,), (1,)), ((), ())),
                             preferred_element_type=jnp.float32)
    d2 = b2_ref[:, :] - 2.0 * ab
    m = jnp.min(d2, axis=1, keepdims=True)
    lane = jax.lax.broadcasted_iota(jnp.int32, (_B, _K), 1)
    idx = jnp.min(jnp.where(d2 <= m, lane, _K), axis=1)
    out_ref[0, 0, :] = idx


_assign = pl.pallas_call(
    _assign_body,
    grid=(_NBLK,),
    in_specs=[
        pl.BlockSpec((_B, _F), lambda b: (b, 0)),
        pl.BlockSpec((_K, _F), lambda b: (0, 0)),
        pl.BlockSpec((1, _K), lambda b: (0, 0)),
    ],
    out_specs=pl.BlockSpec((1, 1, _B), lambda b: (b, 0, 0)),
    out_shape=jax.ShapeDtypeStruct((_NBLK, 1, _B), jnp.int32),
)


_NSUB = 4
_SW = _NSUB * 128


def _reduce_body(xp_ref, perm_ref, ks_ref, out_ref, pwin, kwin, acc, sem,
                 *xbufs):
    info = plsc.get_sparse_core_info()
    nc = info.num_cores
    wid = lax.axis_index("s") * nc + lax.axis_index("c")
    half = wid // 16
    j = wid % 16
    lo = half * _HALF + _chunk_lo(j)
    length = jnp.where(j < 15, _chunk_lo(j + 1), _HALF) - _chunk_lo(j)

    zeros16 = jnp.zeros((16,), jnp.float32)
    one16 = jnp.where(lax.iota(jnp.int32, 16) == 0, 1.0, 0.0)

    def zero_body(i, _):
        acc[pl.ds(i * 16, 16)] = zeros16
        return 0

    lax.fori_loop(0, (_K * _R) // 16, zero_body, 0)

    nsw = (length + _SW - 1) // _SW

    def sw_body(sw, _):
        start = lo + sw * _SW
        rows_sw = jnp.minimum(_SW, length - sw * _SW)
        nsub = rows_sw // 128
        pltpu.sync_copy(ks_ref.at[pl.ds(start, _SW)], kwin)
        for s in range(_NSUB):
            @pl.when(s < nsub)
            def _(s=s):
                pltpu.sync_copy(
                    perm_ref.at[pl.ds(start + s * 128, 128)], pwin.at[s])
                pltpu.async_copy(xp_ref.at[pwin.at[s]], xbufs[s], sem)
        for s in range(_NSUB):
            @pl.when(s < nsub)
            def _(s=s):
                pltpu.make_async_copy(
                    xp_ref.at[pwin.at[s]], xbufs[s], sem).wait()
        for s in range(_NSUB):
            @pl.when(s < nsub)
            def _(s=s):
                def grp_body(g, _):
                    kvec = kwin[pl.ds(s * 128 + g * 16, 16)]
                    for t in range(16):
                        k = kvec[t]
                        base = k * _R
                        row = g * 16 + t
                        for h in range(_F // 16):
                            av = acc[pl.ds(base + h * 16, 16)]
                            xv = xbufs[s][row, pl.ds(h * 16, 16)]
                            acc[pl.ds(base + h * 16, 16)] = av + xv
                        cv = acc[pl.ds(base + _F, 16)]
                        acc[pl.ds(base + _F, 16)] = cv + one16
                    return 0

                lax.fori_loop(0, 8, grp_body, 0)
        return 0

    lax.fori_loop(0, nsw, sw_body, 0)
    pltpu.sync_copy(acc, out_ref.at[wid])


_reduce = functools.partial(
    pl.kernel,
    mesh=plsc.VectorSubcoreMesh(core_axis_name="c", subcore_axis_name="s"),
    out_type=jax.ShapeDtypeStruct((_NW, _K * _R), jnp.float32),
    scratch_types=[
        pltpu.VMEM((_NSUB, 128), jnp.int32),
        pltpu.VMEM((_SW,), jnp.int32),
        pltpu.VMEM((_K * _R,), jnp.float32),
        pltpu.SemaphoreType.DMA,
    ] + [pltpu.VMEM((128, 128), jnp.float32) for _ in range(_NSUB)],
)(_reduce_body)


def _fold_body(planes_ref, c_ref, cnew_ref):
    s = planes_ref[0, :, :]
    for c in range(1, _NW):
        s = s + planes_ref[c, :, :]
    counts = s[:, _F:_F + 1]
    sums = s[:, 0:_F]
    means = sums / jnp.maximum(counts, 1.0)
    cnew_ref[:, :] = jnp.where(counts > 0.0, means, c_ref[:, :])


_fold = pl.pallas_call(
    _fold_body,
    in_specs=[
        pl.BlockSpec((_NW, _K, _R), lambda: (0, 0, 0)),
        pl.BlockSpec((_K, _F), lambda: (0, 0)),
    ],
    out_specs=pl.BlockSpec((_K, _F), lambda: (0, 0)),
    out_shape=jax.ShapeDtypeStruct((_K, _F), jnp.float32),
)


@jax.jit
def kernel(x, centroids):
    xb = x.astype(jnp.bfloat16)
    iota = jnp.arange(_N, dtype=jnp.int32)
    xp = jnp.pad(x, ((0, _PAD), (0, 128 - _F)))
    c = centroids
    asg = None
    for it in range(_ITER):
        cb = c.astype(jnp.bfloat16)
        b2 = jnp.sum(c * c, axis=1)[None, :]
        asg = _assign(xb, cb, b2).reshape(_N)
        if it < _ITER - 1:
            ks_sorted, perm = jax.lax.sort((asg, iota), num_keys=1)
            permp = jnp.concatenate([perm, jnp.full((_PAD,), _N, jnp.int32)])
            ksp = jnp.concatenate(
                [ks_sorted, jnp.zeros((_PAD,), jnp.int32)], axis=0)
            planes = _reduce(xp, permp, ksp)
            c = _fold(planes.reshape(_NW, _K, _R), c)
    return asg


# speedup vs baseline: 1.9149x; 1.0360x over previous
import functools

import jax
import jax.numpy as jnp
from jax import lax
from jax.experimental import pallas as pl
from jax.experimental.pallas import tpu as pltpu
from jax.experimental.pallas import tpu_sc as plsc

_N = 131072
_F = 32
_K = 512
_ITER = 10
_B = 2048
_NBLK = _N // _B

_R = 48
_NW = 32
_HALF = _N // 2
_WIN = 960
_PAD = 1024


def _chunk_lo(j):
    return jnp.where(j <= 11, 4224 * j, 46464 + 3840 * (j - 11))


def _assign_body(xb_ref, cb_ref, b2_ref, out_ref):
    ab = jax.lax.dot_general(xb_ref[:, :], cb_ref[:, :],
                             (((1,), (1,)), ((), ())),
                             preferred_element_type=jnp.float32)
    d2 = b2_ref[:, :] - 2.0 * ab
    idx = jnp.argmin(d2, axis=1).astype(jnp.int32)
    out_ref[0, 0, :] = idx


_assign = pl.pallas_call(
    _assign_body,
    grid=(_NBLK,),
    in_specs=[
        pl.BlockSpec((_B, _F), lambda b: (b, 0)),
        pl.BlockSpec((_K, _F), lambda b: (0, 0)),
        pl.BlockSpec((1, _K), lambda b: (0, 0)),
    ],
    out_specs=pl.BlockSpec((1, 1, _B), lambda b: (b, 0, 0)),
    out_shape=jax.ShapeDtypeStruct((_NBLK, 1, _B), jnp.int32),
)


_NSUB = 4
_SW = _NSUB * 128


def _reduce_body(xp_ref, perm_ref, ks_ref, out_ref, pwin, kwin, acc, sem,
                 *xbufs):
    info = plsc.get_sparse_core_info()
    nc = info.num_cores
    wid = lax.axis_index("s") * nc + lax.axis_index("c")
    half = wid // 16
    j = wid % 16
    lo = half * _HALF + _chunk_lo(j)
    length = jnp.where(j < 15, _chunk_lo(j + 1), _HALF) - _chunk_lo(j)

    zeros16 = jnp.zeros((16,), jnp.float32)
    one16 = jnp.where(lax.iota(jnp.int32, 16) == 0, 1.0, 0.0)

    def zero_body(i, _):
        acc[pl.ds(i * 16, 16)] = zeros16
        return 0

    lax.fori_loop(0, (_K * _R) // 16, zero_body, 0)

    nsw = (length + _SW - 1) // _SW

    def sw_body(sw, _):
        start = lo + sw * _SW
        rows_sw = jnp.minimum(_SW, length - sw * _SW)
        nsub = rows_sw // 128
        pltpu.sync_copy(ks_ref.at[pl.ds(start, _SW)], kwin)
        for s in range(_NSUB):
            @pl.when(s < nsub)
            def _(s=s):
                pltpu.sync_copy(
                    perm_ref.at[pl.ds(start + s * 128, 128)], pwin.at[s])
                pltpu.async_copy(xp_ref.at[pwin.at[s]], xbufs[s], sem)
        for s in range(_NSUB):
            @pl.when(s < nsub)
            def _(s=s):
                pltpu.make_async_copy(
                    xp_ref.at[pwin.at[s]], xbufs[s], sem).wait()
        for s in range(_NSUB):
            @pl.when(s < nsub)
            def _(s=s):
                def grp_body(g, _):
                    kvec = kwin[pl.ds(s * 128 + g * 16, 16)]
                    for t in range(16):
                        k = kvec[t]
                        base = k * _R
                        row = g * 16 + t
                        for h in range(_F // 16):
                            av = acc[pl.ds(base + h * 16, 16)]
                            xv = xbufs[s][row, pl.ds(h * 16, 16)]
                            acc[pl.ds(base + h * 16, 16)] = av + xv
                        cv = acc[pl.ds(base + _F, 16)]
                        acc[pl.ds(base + _F, 16)] = cv + one16
                    return 0

                lax.fori_loop(0, 8, grp_body, 0)
        return 0

    lax.fori_loop(0, nsw, sw_body, 0)
    pltpu.sync_copy(acc, out_ref.at[wid])


_reduce = functools.partial(
    pl.kernel,
    mesh=plsc.VectorSubcoreMesh(core_axis_name="c", subcore_axis_name="s"),
    out_type=jax.ShapeDtypeStruct((_NW, _K * _R), jnp.float32),
    scratch_types=[
        pltpu.VMEM((_NSUB, 128), jnp.int32),
        pltpu.VMEM((_SW,), jnp.int32),
        pltpu.VMEM((_K * _R,), jnp.float32),
        pltpu.SemaphoreType.DMA,
    ] + [pltpu.VMEM((128, 128), jnp.float32) for _ in range(_NSUB)],
)(_reduce_body)


def _fold_body(planes_ref, c_ref, cnew_ref):
    s = planes_ref[0, :, :]
    for c in range(1, _NW):
        s = s + planes_ref[c, :, :]
    counts = s[:, _F:_F + 1]
    sums = s[:, 0:_F]
    means = sums / jnp.maximum(counts, 1.0)
    cnew_ref[:, :] = jnp.where(counts > 0.0, means, c_ref[:, :])


_fold = pl.pallas_call(
    _fold_body,
    in_specs=[
        pl.BlockSpec((_NW, _K, _R), lambda: (0, 0, 0)),
        pl.BlockSpec((_K, _F), lambda: (0, 0)),
    ],
    out_specs=pl.BlockSpec((_K, _F), lambda: (0, 0)),
    out_shape=jax.ShapeDtypeStruct((_K, _F), jnp.float32),
)


@jax.jit
def kernel(x, centroids):
    xb = x.astype(jnp.bfloat16)
    iota = jnp.arange(_N, dtype=jnp.int32)
    xp = jnp.pad(x, ((0, _PAD), (0, 128 - _F)))
    c = centroids
    asg = None
    for it in range(_ITER):
        cb = c.astype(jnp.bfloat16)
        b2 = jnp.sum(c * c, axis=1)[None, :]
        asg = _assign(xb, cb, b2).reshape(_N)
        if it < _ITER - 1:
            ks_sorted, perm = jax.lax.sort((asg, iota), num_keys=1)
            permp = jnp.concatenate([perm, jnp.full((_PAD,), _N, jnp.int32)])
            ksp = jnp.concatenate(
                [ks_sorted, jnp.zeros((_PAD,), jnp.int32)], axis=0)
            planes = _reduce(xp, permp, ksp)
            c = _fold(planes.reshape(_NW, _K, _R), c)
    return asg
